# baseline (speedup 1.0000x reference)
import jax
import jax.numpy as jnp
from jax import lax
from jax.experimental import pallas as pl
from jax.experimental.pallas import tpu as pltpu
from jax.experimental.pallas import tpu_sc as plsc

N = 50000
D = 256
G = 128
LANES = 16
CG = D // LANES
K = 64
RB = 128
NEG_INF = float("-inf")

_info = plsc.get_sparse_core_info()
NC, NS = _info.num_cores, _info.num_subcores
NW = NC * NS
G_SC = NW
G_TC = G - G_SC
STARTS_PAD = G + LANES


def _sc_body(x_hbm, starts_hbm, out_hbm, starts_v, buf0, buf1,
             out_v, sem0, sem1):
    wid = lax.axis_index("s") * NC + lax.axis_index("c")

    pltpu.sync_copy(starts_hbm, starts_v)
    win = starts_v[pl.ds(wid, LANES)]
    s = win[0]
    e = win[1]
    s_al = (s // 8) * 8
    nch = (e - s_al + (K - 1)) // K
    npair = (nch + 1) // 2

    def chunk_base(ci):
        return pl.multiple_of(jnp.minimum(s_al + ci * K, N - K), 8)

    def start_copy(ci, buf, sem):
        src = x_hbm.at[pl.ds(chunk_base(ci), K), :]
        pltpu.make_async_copy(src, buf, sem).start()

    def wait_copy(ci, buf, sem):
        src = x_hbm.at[pl.ds(chunk_base(ci), K), :]
        pltpu.make_async_copy(src, buf, sem).wait()

    def reduce_chunk(accs, ci, buf):
        base = chunk_base(ci)
        j_lo = jnp.maximum(s - base, 0)
        j_hi = jnp.clip(e - base, 0, K)
        j_hi = jnp.where(ci < nch, j_hi, 0)

        def row_body(j, accs):
            return tuple(
                jnp.maximum(accs[c], buf[j, c * LANES:(c + 1) * LANES])
                for c in range(CG)
            )

        return lax.fori_loop(j_lo, j_hi, row_body, accs)

    @pl.when(nch > 0)
    def _():
        start_copy(0, buf0, sem0)

    def pair_body(p, accs):
        c0 = 2 * p
        @pl.when(c0 + 1 < nch)
        def _():
            start_copy(c0 + 1, buf1, sem1)
        wait_copy(c0, buf0, sem0)
        accs = reduce_chunk(accs, c0, buf0)
        @pl.when(c0 + 2 < nch)
        def _():
            start_copy(c0 + 2, buf0, sem0)
        @pl.when(c0 + 1 < nch)
        def _():
            wait_copy(c0 + 1, buf1, sem1)
        accs = reduce_chunk(accs, c0 + 1, buf1)
        return accs

    acc0 = tuple(jnp.full((LANES,), NEG_INF, jnp.float32) for _ in range(CG))
    accs = lax.fori_loop(0, npair, pair_body, acc0)
    for c in range(CG):
        out_v[0, c * LANES:(c + 1) * LANES] = accs[c]

    pltpu.sync_copy(out_v, out_hbm.at[wid])


def _tc_body(starts_s, x_any, out_v, buf0, buf1, sem0, sem1):
    def seg_body(g, _):
        s = starts_s[G_SC + g]
        e = starts_s[G_SC + g + 1]
        s_al = (s // 8) * 8
        nch = (e - s_al + (RB - 1)) // RB
        npair = (nch + 1) // 2

        def chunk_base(ci):
            return pl.multiple_of(jnp.minimum(s_al + ci * RB, N - RB), 8)

        def start_copy(ci, buf, sem):
            src = x_any.at[pl.ds(chunk_base(ci), RB), :]
            pltpu.make_async_copy(src, buf, sem).start()

        def wait_copy(ci, buf, sem):
            src = x_any.at[pl.ds(chunk_base(ci), RB), :]
            pltpu.make_async_copy(src, buf, sem).wait()

        def reduce_chunk(acc, ci, buf):
            base = chunk_base(ci)
            ok = ci < nch
            rows = base + lax.broadcasted_iota(jnp.int32, (RB, 1), 0)
            valid = jnp.logical_and(
                jnp.logical_and(rows >= s, rows < e), ok)
            xm = jnp.where(valid, buf[...], NEG_INF)
            return jnp.maximum(acc, jnp.max(xm.reshape(RB // 8, 8, D), axis=0))

        @pl.when(nch > 0)
        def _():
            start_copy(0, buf0, sem0)

        def pair_body(p, acc):
            c0 = 2 * p
            @pl.when(c0 + 1 < nch)
            def _():
                start_copy(c0 + 1, buf1, sem1)
            wait_copy(c0, buf0, sem0)
            acc = reduce_chunk(acc, c0, buf0)
            @pl.when(c0 + 2 < nch)
            def _():
                start_copy(c0 + 2, buf0, sem0)
            @pl.when(c0 + 1 < nch)
            def _():
                wait_copy(c0 + 1, buf1, sem1)
            acc = reduce_chunk(acc, c0 + 1, buf1)
            return acc

        acc = jnp.full((8, D), NEG_INF, jnp.float32)
        acc = lax.fori_loop(0, npair, pair_body, acc)
        out_v[pl.ds(g, 1), :] = jnp.max(acc, axis=0, keepdims=True)
        return 0

    lax.fori_loop(0, G_TC, seg_body, 0)


@jax.jit
def kernel(x, batch):
    starts = jnp.searchsorted(
        batch, jnp.arange(G + 1, dtype=jnp.int32), method="compare_all"
    ).astype(jnp.int32)
    starts = jnp.concatenate(
        [starts, jnp.full((STARTS_PAD - (G + 1),), N, jnp.int32)])

    sc_fn = pl.kernel(
        _sc_body,
        out_type=jax.ShapeDtypeStruct((NW, 1, D), jnp.float32),
        mesh=plsc.VectorSubcoreMesh(core_axis_name="c", subcore_axis_name="s"),
        scratch_types=[
            pltpu.VMEM((STARTS_PAD,), jnp.int32),
            pltpu.VMEM((K, D), jnp.float32),
            pltpu.VMEM((K, D), jnp.float32),
            pltpu.VMEM((1, D), jnp.float32),
            pltpu.SemaphoreType.DMA,
            pltpu.SemaphoreType.DMA,
        ],
    )
    sc_out = sc_fn(x, starts)

    tc_out = pl.pallas_call(
        _tc_body,
        out_shape=jax.ShapeDtypeStruct((G_TC, D), jnp.float32),
        in_specs=[
            pl.BlockSpec(memory_space=pltpu.SMEM),
            pl.BlockSpec(memory_space=pl.ANY),
        ],
        out_specs=pl.BlockSpec((G_TC, D), lambda: (0, 0)),
        scratch_shapes=[
            pltpu.VMEM((RB, D), jnp.float32),
            pltpu.VMEM((RB, D), jnp.float32),
            pltpu.SemaphoreType.DMA,
            pltpu.SemaphoreType.DMA,
        ],
    )(starts, x)

    return jnp.concatenate([sc_out.reshape(G_SC, D), tc_out], axis=0)
